# baseline (speedup 1.0000x reference)
import functools

import jax
import jax.numpy as jnp
from jax.experimental import pallas as pl

_SA_CHANNELS = ((32, 32, 64), (64, 64, 128), (128, 128, 256), (256, 256, 512))
_NUM_CENTROIDS = (2048, 512, 128, 32)
_RADIUS = (0.1, 0.2, 0.4, 0.8)
_MAX_NEIGHBORS = (32, 32, 32, 32)
_FP_CHANNELS = ((256, 256), (256, 256), (256, 128), (128, 128, 128))
_FP_NEIGHBORS = (3, 3, 3, 3)


def _gather(x, idx):
    return jax.vmap(lambda xi, ii: xi[ii])(x, idx)


def _pairwise_d2(a, b):
    aa = jnp.sum(a * a, -1)[:, :, None]
    bb = jnp.sum(b * b, -1)[:, None, :]
    ab = jnp.einsum('bmd,bnd->bmn', a, b)
    return jnp.maximum(aa + bb - 2.0 * ab, 0.0)


def _fps(xyz, m):
    xyz = jax.lax.stop_gradient(xyz)
    B, N, _ = xyz.shape

    def body(i, state):
        idxs, dists, last = state
        lp = jnp.take_along_axis(xyz, last[:, None, None], axis=1)
        d = jnp.sum((xyz - lp) ** 2, axis=-1)
        dists = jnp.minimum(dists, d)
        nxt = jnp.argmax(dists, axis=-1).astype(jnp.int32)
        idxs = idxs.at[:, i].set(nxt)
        return idxs, dists, nxt

    idxs = jnp.zeros((B, m), jnp.int32)
    dists = jnp.full((B, N), 1e10, jnp.float32)
    last = jnp.zeros((B,), jnp.int32)
    idxs, _, _ = jax.lax.fori_loop(1, m, body, (idxs, dists, last))
    return idxs


def _ball_query(xyz, new_xyz, radius, K):
    d2 = _pairwise_d2(jax.lax.stop_gradient(new_xyz), jax.lax.stop_gradient(xyz))
    N = xyz.shape[1]
    score = jnp.where(d2 <= radius * radius, jnp.arange(N, dtype=jnp.int32)[None, None, :], N)
    idx = jnp.sort(score, axis=-1)[:, :, :K]
    first = idx[:, :, :1]
    idx = jnp.where(idx >= N, first, idx)
    return idx


def _set_abstraction(xyz, feat, params, m, radius, K):
    cent_idx = _fps(xyz, m)
    new_xyz = _gather(xyz, cent_idx)
    nbr_idx = _ball_query(xyz, new_xyz, radius, K)
    g_xyz = _gather(xyz, nbr_idx) - new_xyz[:, :, None, :]
    if feat is not None:
        g_feat = _gather(feat, nbr_idx)
        g = jnp.concatenate([g_xyz, g_feat], axis=-1)
    else:
        g = g_xyz
    for W, b in params:
        g = jax.nn.relu(g @ W + b)
    new_feat = jnp.max(g, axis=2)
    return new_xyz, new_feat


def _feature_propagation(dense_xyz, sparse_xyz, dense_feat, sparse_feat, params, k):
    d2 = _pairwise_d2(jax.lax.stop_gradient(dense_xyz), jax.lax.stop_gradient(sparse_xyz))
    neg, idx = jax.lax.top_k(-d2, k)
    dist = jnp.maximum(-neg, 0.0)
    w = 1.0 / (dist + 1e-8)
    w = w / jnp.sum(w, axis=-1, keepdims=True)
    nbr = _gather(sparse_feat, idx)
    interp = jnp.sum(nbr * w[..., None], axis=2)
    x = interp if dense_feat is None else jnp.concatenate([interp, dense_feat], axis=-1)
    for W, b in params:
        x = jax.nn.relu(x @ W + b)
    return x


def _identity_pallas(x):
    def k(x_ref, o_ref):
        o_ref[...] = x_ref[...]
    return pl.pallas_call(
        k, out_shape=jax.ShapeDtypeStruct(x.shape, x.dtype))(x)


def kernel(points, sa_params, fp_params):
    xyz = jnp.transpose(points, (0, 2, 1))
    feat = None
    xyz_list = [xyz]
    feat_list = [None]
    for i in range(len(_SA_CHANNELS)):
        xyz, feat = _set_abstraction(xyz, feat, sa_params[i], _NUM_CENTROIDS[i],
                                     _RADIUS[i], _MAX_NEIGHBORS[i])
        xyz_list.append(xyz)
        feat_list.append(feat)
    fp_feat = feat_list[-1]
    for i in range(len(_FP_CHANNELS)):
        fp_feat = _feature_propagation(xyz_list[-2 - i], xyz_list[-1 - i],
                                       feat_list[-2 - i], fp_feat, fp_params[i],
                                       _FP_NEIGHBORS[i])
    fp_feat = _identity_pallas(fp_feat)
    return jnp.transpose(fp_feat, (0, 2, 1))


# speedup vs baseline: 1.6254x; 1.6254x over previous
import functools

import jax
import jax.numpy as jnp
from jax.experimental import pallas as pl

_SA_CHANNELS = ((32, 32, 64), (64, 64, 128), (128, 128, 256), (256, 256, 512))
_NUM_CENTROIDS = (2048, 512, 128, 32)
_RADIUS = (0.1, 0.2, 0.4, 0.8)
_MAX_NEIGHBORS = (32, 32, 32, 32)
_FP_CHANNELS = ((256, 256), (256, 256), (256, 128), (128, 128, 128))
_FP_NEIGHBORS = (3, 3, 3, 3)


def _gather(x, idx):
    return jax.vmap(lambda xi, ii: xi[ii])(x, idx)


def _pairwise_d2(a, b):
    aa = jnp.sum(a * a, -1)[:, :, None]
    bb = jnp.sum(b * b, -1)[:, None, :]
    ab = jnp.einsum('bmd,bnd->bmn', a, b)
    return jnp.maximum(aa + bb - 2.0 * ab, 0.0)


def _fps_kernel(m, N, B, S, L, x_ref, o_ref):
    x = x_ref[:, 0]
    y = x_ref[:, 1]
    z = x_ref[:, 2]
    ii = (jax.lax.broadcasted_iota(jnp.int32, (B, S, L), 1) * L
          + jax.lax.broadcasted_iota(jnp.int32, (B, S, L), 2))
    o_ref[0:1, :] = jnp.zeros((1, B), jnp.int32)
    dists0 = jnp.full((B, S, L), 1e10, jnp.float32)
    last0 = jnp.zeros((B, 1, 1), jnp.int32)

    def body(i, carry):
        dists, last = carry
        sel = ii == last
        lx = jnp.sum(jnp.where(sel, x, 0.0), axis=(1, 2), keepdims=True)
        ly = jnp.sum(jnp.where(sel, y, 0.0), axis=(1, 2), keepdims=True)
        lz = jnp.sum(jnp.where(sel, z, 0.0), axis=(1, 2), keepdims=True)
        d = (x - lx) ** 2 + (y - ly) ** 2 + (z - lz) ** 2
        dists = jnp.minimum(dists, d)
        mx = jnp.max(dists, axis=(1, 2), keepdims=True)
        cand = jnp.where(dists == mx, ii, N)
        nxt = jnp.min(cand, axis=(1, 2), keepdims=True)
        o_ref[pl.ds(i, 1), :] = nxt.reshape(1, B)
        return dists, nxt

    jax.lax.fori_loop(1, m, body, (dists0, last0))


def _fps(xyz, m, interpret=False):
    xyz = jax.lax.stop_gradient(xyz)
    B, N, _ = xyz.shape
    L = 128
    S = N // L
    xyzf = jnp.transpose(xyz, (0, 2, 1)).reshape(B, 3, S, L)
    idx_t = pl.pallas_call(
        functools.partial(_fps_kernel, m, N, B, S, L),
        out_shape=jax.ShapeDtypeStruct((m, B), jnp.int32),
        interpret=interpret,
    )(xyzf)
    return idx_t.T


def _ball_query(xyz, new_xyz, radius, K):
    d2 = _pairwise_d2(jax.lax.stop_gradient(new_xyz), jax.lax.stop_gradient(xyz))
    N = xyz.shape[1]
    score = jnp.where(d2 <= radius * radius, jnp.arange(N, dtype=jnp.int32)[None, None, :], N)
    idx = jnp.sort(score, axis=-1)[:, :, :K]
    first = idx[:, :, :1]
    idx = jnp.where(idx >= N, first, idx)
    return idx


def _set_abstraction(xyz, feat, params, m, radius, K):
    cent_idx = _fps(xyz, m)
    new_xyz = _gather(xyz, cent_idx)
    nbr_idx = _ball_query(xyz, new_xyz, radius, K)
    g_xyz = _gather(xyz, nbr_idx) - new_xyz[:, :, None, :]
    if feat is not None:
        g_feat = _gather(feat, nbr_idx)
        g = jnp.concatenate([g_xyz, g_feat], axis=-1)
    else:
        g = g_xyz
    for W, b in params:
        g = jax.nn.relu(g @ W + b)
    new_feat = jnp.max(g, axis=2)
    return new_xyz, new_feat


def _feature_propagation(dense_xyz, sparse_xyz, dense_feat, sparse_feat, params, k):
    d2 = _pairwise_d2(jax.lax.stop_gradient(dense_xyz), jax.lax.stop_gradient(sparse_xyz))
    neg, idx = jax.lax.top_k(-d2, k)
    dist = jnp.maximum(-neg, 0.0)
    w = 1.0 / (dist + 1e-8)
    w = w / jnp.sum(w, axis=-1, keepdims=True)
    nbr = _gather(sparse_feat, idx)
    interp = jnp.sum(nbr * w[..., None], axis=2)
    x = interp if dense_feat is None else jnp.concatenate([interp, dense_feat], axis=-1)
    for W, b in params:
        x = jax.nn.relu(x @ W + b)
    return x


def _identity_pallas(x):
    def k(x_ref, o_ref):
        o_ref[...] = x_ref[...]
    return pl.pallas_call(
        k, out_shape=jax.ShapeDtypeStruct(x.shape, x.dtype))(x)


def kernel(points, sa_params, fp_params):
    xyz = jnp.transpose(points, (0, 2, 1))
    feat = None
    xyz_list = [xyz]
    feat_list = [None]
    for i in range(len(_SA_CHANNELS)):
        xyz, feat = _set_abstraction(xyz, feat, sa_params[i], _NUM_CENTROIDS[i],
                                     _RADIUS[i], _MAX_NEIGHBORS[i])
        xyz_list.append(xyz)
        feat_list.append(feat)
    fp_feat = feat_list[-1]
    for i in range(len(_FP_CHANNELS)):
        fp_feat = _feature_propagation(xyz_list[-2 - i], xyz_list[-1 - i],
                                       feat_list[-2 - i], fp_feat, fp_params[i],
                                       _FP_NEIGHBORS[i])
    fp_feat = _identity_pallas(fp_feat)
    return jnp.transpose(fp_feat, (0, 2, 1))


# speedup vs baseline: 2.5446x; 1.5655x over previous
import functools

import jax
import jax.numpy as jnp
from jax.experimental import pallas as pl

_SA_CHANNELS = ((32, 32, 64), (64, 64, 128), (128, 128, 256), (256, 256, 512))
_NUM_CENTROIDS = (2048, 512, 128, 32)
_RADIUS = (0.1, 0.2, 0.4, 0.8)
_MAX_NEIGHBORS = (32, 32, 32, 32)
_FP_CHANNELS = ((256, 256), (256, 256), (256, 128), (128, 128, 128))
_FP_NEIGHBORS = (3, 3, 3, 3)


def _gather(x, idx):
    return jax.vmap(lambda xi, ii: xi[ii])(x, idx)


def _pairwise_d2(a, b):
    aa = jnp.sum(a * a, -1)[:, :, None]
    bb = jnp.sum(b * b, -1)[:, None, :]
    ab = jnp.einsum('bmd,bnd->bmn', a, b)
    return jnp.maximum(aa + bb - 2.0 * ab, 0.0)


def _fps_kernel(m, N, B, S, L, x_ref, o_ref):
    x = x_ref[:, 0]
    y = x_ref[:, 1]
    z = x_ref[:, 2]
    ii = (jax.lax.broadcasted_iota(jnp.int32, (B, S, L), 1) * L
          + jax.lax.broadcasted_iota(jnp.int32, (B, S, L), 2))
    o_ref[0:1, :] = jnp.zeros((1, B), jnp.int32)
    dists0 = jnp.full((B, S, L), 1e10, jnp.float32)
    last0 = jnp.zeros((B, 1, 1), jnp.int32)

    def body(i, carry):
        dists, last = carry
        sel = ii == last
        lx = jnp.sum(jnp.where(sel, x, 0.0), axis=(1, 2), keepdims=True)
        ly = jnp.sum(jnp.where(sel, y, 0.0), axis=(1, 2), keepdims=True)
        lz = jnp.sum(jnp.where(sel, z, 0.0), axis=(1, 2), keepdims=True)
        d = (x - lx) ** 2 + (y - ly) ** 2 + (z - lz) ** 2
        dists = jnp.minimum(dists, d)
        mx = jnp.max(dists, axis=(1, 2), keepdims=True)
        cand = jnp.where(dists == mx, ii, N)
        nxt = jnp.min(cand, axis=(1, 2), keepdims=True)
        o_ref[pl.ds(i, 1), :] = nxt.reshape(1, B)
        return dists, nxt

    jax.lax.fori_loop(1, m, body, (dists0, last0))


def _fps(xyz, m, interpret=False):
    xyz = jax.lax.stop_gradient(xyz)
    B, N, _ = xyz.shape
    L = 128
    S = N // L
    xyzf = jnp.transpose(xyz, (0, 2, 1)).reshape(B, 3, S, L)
    idx_t = pl.pallas_call(
        functools.partial(_fps_kernel, m, N, B, S, L),
        out_shape=jax.ShapeDtypeStruct((m, B), jnp.int32),
        interpret=interpret,
    )(xyzf)
    return idx_t.T


def _ball_query_kernel(N, K, r2, xyz_ref, new_ref, o_ref):
    xyz_t = xyz_ref[0]
    new = new_ref[0]
    bb = jnp.sum(xyz_t * xyz_t, axis=0, keepdims=True)
    aa = jnp.sum(new * new, axis=1, keepdims=True)
    ab = jnp.dot(new, xyz_t, preferred_element_type=jnp.float32)
    d2 = jnp.maximum(aa + bb - 2.0 * ab, 0.0)
    mask = (d2 <= r2).astype(jnp.int32)
    c = mask
    s = 1
    while s < N:
        shifted = jnp.concatenate(
            [jnp.zeros((c.shape[0], s), c.dtype), c[:, :N - s]], axis=1)
        c = c + shifted
        s *= 2
    cols = []
    for k in range(K):
        cols.append(jnp.sum((c <= k).astype(jnp.int32), axis=1, keepdims=True))
    idx = jnp.concatenate(cols, axis=1)
    first = idx[:, 0:1]
    idx = jnp.where(idx >= N, first, idx)
    o_ref[0] = idx


def _ball_query(xyz, new_xyz, radius, K, interpret=False):
    xyz = jax.lax.stop_gradient(xyz)
    new_xyz = jax.lax.stop_gradient(new_xyz)
    B, N, _ = xyz.shape
    M = new_xyz.shape[1]
    Mb = min(M, 256)
    xyz_t = jnp.transpose(xyz, (0, 2, 1))
    return pl.pallas_call(
        functools.partial(_ball_query_kernel, N, K, radius * radius),
        grid=(B, M // Mb),
        in_specs=[
            pl.BlockSpec((1, 3, N), lambda b, mb: (b, 0, 0)),
            pl.BlockSpec((1, Mb, 3), lambda b, mb: (b, mb, 0)),
        ],
        out_specs=pl.BlockSpec((1, Mb, K), lambda b, mb: (b, mb, 0)),
        out_shape=jax.ShapeDtypeStruct((B, M, K), jnp.int32),
        interpret=interpret,
    )(xyz_t, new_xyz)


def _set_abstraction(xyz, feat, params, m, radius, K):
    cent_idx = _fps(xyz, m)
    new_xyz = _gather(xyz, cent_idx)
    nbr_idx = _ball_query(xyz, new_xyz, radius, K)
    g_xyz = _gather(xyz, nbr_idx) - new_xyz[:, :, None, :]
    if feat is not None:
        g_feat = _gather(feat, nbr_idx)
        g = jnp.concatenate([g_xyz, g_feat], axis=-1)
    else:
        g = g_xyz
    for W, b in params:
        g = jax.nn.relu(g @ W + b)
    new_feat = jnp.max(g, axis=2)
    return new_xyz, new_feat


def _feature_propagation(dense_xyz, sparse_xyz, dense_feat, sparse_feat, params, k):
    d2 = _pairwise_d2(jax.lax.stop_gradient(dense_xyz), jax.lax.stop_gradient(sparse_xyz))
    neg, idx = jax.lax.top_k(-d2, k)
    dist = jnp.maximum(-neg, 0.0)
    w = 1.0 / (dist + 1e-8)
    w = w / jnp.sum(w, axis=-1, keepdims=True)
    nbr = _gather(sparse_feat, idx)
    interp = jnp.sum(nbr * w[..., None], axis=2)
    x = interp if dense_feat is None else jnp.concatenate([interp, dense_feat], axis=-1)
    for W, b in params:
        x = jax.nn.relu(x @ W + b)
    return x


def _identity_pallas(x):
    def k(x_ref, o_ref):
        o_ref[...] = x_ref[...]
    return pl.pallas_call(
        k, out_shape=jax.ShapeDtypeStruct(x.shape, x.dtype))(x)


def kernel(points, sa_params, fp_params):
    xyz = jnp.transpose(points, (0, 2, 1))
    feat = None
    xyz_list = [xyz]
    feat_list = [None]
    for i in range(len(_SA_CHANNELS)):
        xyz, feat = _set_abstraction(xyz, feat, sa_params[i], _NUM_CENTROIDS[i],
                                     _RADIUS[i], _MAX_NEIGHBORS[i])
        xyz_list.append(xyz)
        feat_list.append(feat)
    fp_feat = feat_list[-1]
    for i in range(len(_FP_CHANNELS)):
        fp_feat = _feature_propagation(xyz_list[-2 - i], xyz_list[-1 - i],
                                       feat_list[-2 - i], fp_feat, fp_params[i],
                                       _FP_NEIGHBORS[i])
    fp_feat = _identity_pallas(fp_feat)
    return jnp.transpose(fp_feat, (0, 2, 1))


# speedup vs baseline: 5.8368x; 2.2938x over previous
import functools

import jax
import jax.numpy as jnp
from jax.experimental import pallas as pl

_SA_CHANNELS = ((32, 32, 64), (64, 64, 128), (128, 128, 256), (256, 256, 512))
_NUM_CENTROIDS = (2048, 512, 128, 32)
_RADIUS = (0.1, 0.2, 0.4, 0.8)
_MAX_NEIGHBORS = (32, 32, 32, 32)
_FP_CHANNELS = ((256, 256), (256, 256), (256, 128), (128, 128, 128))
_FP_NEIGHBORS = (3, 3, 3, 3)


def _gather(x, idx):
    return jax.vmap(lambda xi, ii: xi[ii])(x, idx)


def _pairwise_d2(a, b):
    aa = jnp.sum(a * a, -1)[:, :, None]
    bb = jnp.sum(b * b, -1)[:, None, :]
    ab = jnp.einsum('bmd,bnd->bmn', a, b)
    return jnp.maximum(aa + bb - 2.0 * ab, 0.0)


def _fps_kernel(m, N, B, S, L, x_ref, o_ref):
    x = x_ref[:, 0]
    y = x_ref[:, 1]
    z = x_ref[:, 2]
    ii = (jax.lax.broadcasted_iota(jnp.int32, (B, S, L), 1) * L
          + jax.lax.broadcasted_iota(jnp.int32, (B, S, L), 2))
    o_ref[0:1, :] = jnp.zeros((1, B), jnp.int32)
    dists0 = jnp.full((B, S, L), 1e10, jnp.float32)
    last0 = jnp.zeros((B, 1, 1), jnp.int32)

    def body(i, carry):
        dists, last = carry
        sel = ii == last
        lx = jnp.sum(jnp.where(sel, x, 0.0), axis=(1, 2), keepdims=True)
        ly = jnp.sum(jnp.where(sel, y, 0.0), axis=(1, 2), keepdims=True)
        lz = jnp.sum(jnp.where(sel, z, 0.0), axis=(1, 2), keepdims=True)
        d = (x - lx) ** 2 + (y - ly) ** 2 + (z - lz) ** 2
        dists = jnp.minimum(dists, d)
        mx = jnp.max(dists, axis=(1, 2), keepdims=True)
        cand = jnp.where(dists == mx, ii, N)
        nxt = jnp.min(cand, axis=(1, 2), keepdims=True)
        o_ref[pl.ds(i, 1), :] = nxt.reshape(1, B)
        return dists, nxt

    jax.lax.fori_loop(1, m, body, (dists0, last0))


def _fps(xyz, m, interpret=False):
    xyz = jax.lax.stop_gradient(xyz)
    B, N, _ = xyz.shape
    L = 128
    S = N // L
    xyzf = jnp.transpose(xyz, (0, 2, 1)).reshape(B, 3, S, L)
    idx_t = pl.pallas_call(
        functools.partial(_fps_kernel, m, N, B, S, L),
        out_shape=jax.ShapeDtypeStruct((m, B), jnp.int32),
        interpret=interpret,
    )(xyzf)
    return idx_t.T


def _ball_query_kernel(N, K, r2, xyz_ref, new_ref, o_ref):
    xyz_t = xyz_ref[0]
    new = new_ref[0]
    bb = jnp.sum(xyz_t * xyz_t, axis=0, keepdims=True)
    aa = jnp.sum(new * new, axis=1, keepdims=True)
    ab = jnp.dot(new, xyz_t, preferred_element_type=jnp.float32)
    d2 = jnp.maximum(aa + bb - 2.0 * ab, 0.0)
    mask = (d2 <= r2).astype(jnp.int32)
    c = mask
    s = 1
    while s < N:
        shifted = jnp.concatenate(
            [jnp.zeros((c.shape[0], s), c.dtype), c[:, :N - s]], axis=1)
        c = c + shifted
        s *= 2
    cols = []
    for k in range(K):
        cols.append(jnp.sum((c <= k).astype(jnp.int32), axis=1, keepdims=True))
    idx = jnp.concatenate(cols, axis=1)
    first = idx[:, 0:1]
    idx = jnp.where(idx >= N, first, idx)
    o_ref[0] = idx


def _ball_query(xyz, new_xyz, radius, K, interpret=False):
    xyz = jax.lax.stop_gradient(xyz)
    new_xyz = jax.lax.stop_gradient(new_xyz)
    B, N, _ = xyz.shape
    M = new_xyz.shape[1]
    Mb = min(M, 256)
    xyz_t = jnp.transpose(xyz, (0, 2, 1))
    return pl.pallas_call(
        functools.partial(_ball_query_kernel, N, K, radius * radius),
        grid=(B, M // Mb),
        in_specs=[
            pl.BlockSpec((1, 3, N), lambda b, mb: (b, 0, 0)),
            pl.BlockSpec((1, Mb, 3), lambda b, mb: (b, mb, 0)),
        ],
        out_specs=pl.BlockSpec((1, Mb, K), lambda b, mb: (b, mb, 0)),
        out_shape=jax.ShapeDtypeStruct((B, M, K), jnp.int32),
        interpret=interpret,
    )(xyz_t, new_xyz)


def _set_abstraction(xyz, feat, params, m, radius, K):
    cent_idx = _fps(xyz, m)
    new_xyz = _gather(xyz, cent_idx)
    nbr_idx = _ball_query(xyz, new_xyz, radius, K)
    g_xyz = _gather(xyz, nbr_idx) - new_xyz[:, :, None, :]
    if feat is not None:
        g_feat = _gather(feat, nbr_idx)
        g = jnp.concatenate([g_xyz, g_feat], axis=-1)
    else:
        g = g_xyz
    for W, b in params:
        g = jax.nn.relu(g @ W + b)
    new_feat = jnp.max(g, axis=2)
    return new_xyz, new_feat


def _fp_kernel(Ns, k, n_layers, has_dense, *refs):
    dxyz = refs[0][0]
    sxyz_t = refs[1][0]
    sfeat = refs[2][0]
    pos = 3
    dfeat = None
    if has_dense:
        dfeat = refs[pos][0]
        pos += 1
    layers = []
    for _ in range(n_layers):
        layers.append((refs[pos], refs[pos + 1]))
        pos += 2
    o_ref = refs[pos]

    bb = jnp.sum(sxyz_t * sxyz_t, axis=0, keepdims=True)
    aa = jnp.sum(dxyz * dxyz, axis=1, keepdims=True)
    ab = jnp.dot(dxyz, sxyz_t, preferred_element_type=jnp.float32)
    d2 = jnp.maximum(aa + bb - 2.0 * ab, 0.0)
    iota_n = jax.lax.broadcasted_iota(jnp.int32, d2.shape, 1)
    d = d2
    dists, idxs = [], []
    for _ in range(k):
        mn = jnp.min(d, axis=1, keepdims=True)
        ik = jnp.min(jnp.where(d == mn, iota_n, Ns), axis=1, keepdims=True)
        dists.append(mn)
        idxs.append(ik)
        d = jnp.where(iota_n == ik, 1e30, d)
    ws = [1.0 / (mn + 1e-8) for mn in dists]
    denom = ws[0]
    for wk in ws[1:]:
        denom = denom + wk
    wmat = jnp.zeros(d2.shape, jnp.float32)
    for wk, ik in zip(ws, idxs):
        wmat = wmat + jnp.where(iota_n == ik, wk / denom, 0.0)
    x = jnp.dot(wmat, sfeat, preferred_element_type=jnp.float32)
    if dfeat is not None:
        x = jnp.concatenate([x, dfeat], axis=1)
    for W_ref, b_ref in layers:
        x = jnp.maximum(jnp.dot(x, W_ref[...], preferred_element_type=jnp.float32)
                        + b_ref[...], 0.0)
    o_ref[0] = x


def _feature_propagation(dense_xyz, sparse_xyz, dense_feat, sparse_feat, params, k,
                         interpret=False):
    dense_xyz = jax.lax.stop_gradient(dense_xyz)
    sparse_xyz = jax.lax.stop_gradient(sparse_xyz)
    B, Nd, _ = dense_xyz.shape
    Ns = sparse_xyz.shape[1]
    C = sparse_feat.shape[-1]
    Cout = params[-1][0].shape[1]
    Mb = min(Nd, 256)
    sxyz_t = jnp.transpose(sparse_xyz, (0, 2, 1))
    has_dense = dense_feat is not None
    operands = [dense_xyz, sxyz_t, sparse_feat]
    in_specs = [
        pl.BlockSpec((1, Mb, 3), lambda b, mb: (b, mb, 0)),
        pl.BlockSpec((1, 3, Ns), lambda b, mb: (b, 0, 0)),
        pl.BlockSpec((1, Ns, C), lambda b, mb: (b, 0, 0)),
    ]
    if has_dense:
        Cd = dense_feat.shape[-1]
        operands.append(dense_feat)
        in_specs.append(pl.BlockSpec((1, Mb, Cd), lambda b, mb: (b, mb, 0)))
    for W, b in params:
        operands.extend([W, b.reshape(1, -1)])
        in_specs.append(pl.BlockSpec(W.shape, lambda b_, mb: (0, 0)))
        in_specs.append(pl.BlockSpec((1, b.shape[0]), lambda b_, mb: (0, 0)))
    return pl.pallas_call(
        functools.partial(_fp_kernel, Ns, k, len(params), has_dense),
        grid=(B, Nd // Mb),
        in_specs=in_specs,
        out_specs=pl.BlockSpec((1, Mb, Cout), lambda b, mb: (b, mb, 0)),
        out_shape=jax.ShapeDtypeStruct((B, Nd, Cout), jnp.float32),
        interpret=interpret,
    )(*operands)


def _identity_pallas(x):
    def k(x_ref, o_ref):
        o_ref[...] = x_ref[...]
    return pl.pallas_call(
        k, out_shape=jax.ShapeDtypeStruct(x.shape, x.dtype))(x)


def kernel(points, sa_params, fp_params):
    xyz = jnp.transpose(points, (0, 2, 1))
    feat = None
    xyz_list = [xyz]
    feat_list = [None]
    for i in range(len(_SA_CHANNELS)):
        xyz, feat = _set_abstraction(xyz, feat, sa_params[i], _NUM_CENTROIDS[i],
                                     _RADIUS[i], _MAX_NEIGHBORS[i])
        xyz_list.append(xyz)
        feat_list.append(feat)
    fp_feat = feat_list[-1]
    for i in range(len(_FP_CHANNELS)):
        fp_feat = _feature_propagation(xyz_list[-2 - i], xyz_list[-1 - i],
                                       feat_list[-2 - i], fp_feat, fp_params[i],
                                       _FP_NEIGHBORS[i])
    fp_feat = _identity_pallas(fp_feat)
    return jnp.transpose(fp_feat, (0, 2, 1))


# speedup vs baseline: 6.6779x; 1.1441x over previous
import functools

import jax
import jax.numpy as jnp
from jax.experimental import pallas as pl

_SA_CHANNELS = ((32, 32, 64), (64, 64, 128), (128, 128, 256), (256, 256, 512))
_NUM_CENTROIDS = (2048, 512, 128, 32)
_RADIUS = (0.1, 0.2, 0.4, 0.8)
_MAX_NEIGHBORS = (32, 32, 32, 32)
_FP_CHANNELS = ((256, 256), (256, 256), (256, 128), (128, 128, 128))
_FP_NEIGHBORS = (3, 3, 3, 3)


def _gather(x, idx):
    return jax.vmap(lambda xi, ii: xi[ii])(x, idx)


def _pairwise_d2(a, b):
    aa = jnp.sum(a * a, -1)[:, :, None]
    bb = jnp.sum(b * b, -1)[:, None, :]
    ab = jnp.einsum('bmd,bnd->bmn', a, b)
    return jnp.maximum(aa + bb - 2.0 * ab, 0.0)


def _fps_kernel(m, N, B, S, L, x_ref, o_ref):
    x = x_ref[:, 0]
    y = x_ref[:, 1]
    z = x_ref[:, 2]
    ii = (jax.lax.broadcasted_iota(jnp.int32, (B, S, L), 1) * L
          + jax.lax.broadcasted_iota(jnp.int32, (B, S, L), 2))
    o_ref[0:1, :] = jnp.zeros((1, B), jnp.int32)
    dists0 = jnp.full((B, S, L), 1e10, jnp.float32)
    last0 = jnp.zeros((B, 1, 1), jnp.int32)

    def body(i, carry):
        dists, last = carry
        sel = ii == last
        lx = jnp.sum(jnp.where(sel, x, 0.0), axis=(1, 2), keepdims=True)
        ly = jnp.sum(jnp.where(sel, y, 0.0), axis=(1, 2), keepdims=True)
        lz = jnp.sum(jnp.where(sel, z, 0.0), axis=(1, 2), keepdims=True)
        d = (x - lx) ** 2 + (y - ly) ** 2 + (z - lz) ** 2
        dists = jnp.minimum(dists, d)
        mx = jnp.max(dists, axis=(1, 2), keepdims=True)
        cand = jnp.where(dists == mx, ii, N)
        nxt = jnp.min(cand, axis=(1, 2), keepdims=True)
        o_ref[pl.ds(i, 1), :] = nxt.reshape(1, B)
        return dists, nxt

    jax.lax.fori_loop(1, m, body, (dists0, last0))


def _fps(xyz, m, interpret=False):
    xyz = jax.lax.stop_gradient(xyz)
    B, N, _ = xyz.shape
    L = 128
    S = N // L
    xyzf = jnp.transpose(xyz, (0, 2, 1)).reshape(B, 3, S, L)
    idx_t = pl.pallas_call(
        functools.partial(_fps_kernel, m, N, B, S, L),
        out_shape=jax.ShapeDtypeStruct((m, B), jnp.int32),
        interpret=interpret,
    )(xyzf)
    return idx_t.T


def _ball_query_kernel(N, K, r2, xyz_ref, new_ref, o_ref):
    xyz_t = xyz_ref[0]
    new = new_ref[0]
    bb = jnp.sum(xyz_t * xyz_t, axis=0, keepdims=True)
    aa = jnp.sum(new * new, axis=1, keepdims=True)
    ab = jnp.dot(new, xyz_t, preferred_element_type=jnp.float32)
    d2 = jnp.maximum(aa + bb - 2.0 * ab, 0.0)
    mask = (d2 <= r2).astype(jnp.int32)
    c = mask
    s = 1
    while s < N:
        shifted = jnp.concatenate(
            [jnp.zeros((c.shape[0], s), c.dtype), c[:, :N - s]], axis=1)
        c = c + shifted
        s *= 2
    cols = []
    for k in range(K):
        cols.append(jnp.sum((c <= k).astype(jnp.int32), axis=1, keepdims=True))
    idx = jnp.concatenate(cols, axis=1)
    first = idx[:, 0:1]
    idx = jnp.where(idx >= N, first, idx)
    o_ref[0] = idx


def _ball_query(xyz, new_xyz, radius, K, interpret=False):
    xyz = jax.lax.stop_gradient(xyz)
    new_xyz = jax.lax.stop_gradient(new_xyz)
    B, N, _ = xyz.shape
    M = new_xyz.shape[1]
    Mb = min(M, 256)
    xyz_t = jnp.transpose(xyz, (0, 2, 1))
    return pl.pallas_call(
        functools.partial(_ball_query_kernel, N, K, radius * radius),
        grid=(B, M // Mb),
        in_specs=[
            pl.BlockSpec((1, 3, N), lambda b, mb: (b, 0, 0)),
            pl.BlockSpec((1, Mb, 3), lambda b, mb: (b, mb, 0)),
        ],
        out_specs=pl.BlockSpec((1, Mb, K), lambda b, mb: (b, mb, 0)),
        out_shape=jax.ShapeDtypeStruct((B, M, K), jnp.int32),
        interpret=interpret,
    )(xyz_t, new_xyz)


def _sa_tail_kernel(K, n_layers, *refs):
    g = refs[0][0]
    o = refs[1][0]
    Mb, _, C1 = g.shape
    x = jnp.maximum(g - o[:, None, :], 0.0).reshape(Mb * K, C1)
    pos = 2
    for _ in range(n_layers):
        W_ref, b_ref = refs[pos], refs[pos + 1]
        pos += 2
        x = jnp.maximum(jnp.dot(x, W_ref[...], preferred_element_type=jnp.float32)
                        + b_ref[...], 0.0)
    Cout = x.shape[-1]
    refs[pos][0] = jnp.max(x.reshape(Mb, K, Cout), axis=1)


def _sa_tail(G, o, rest, interpret=False):
    B, M, K, C1 = G.shape
    Cout = rest[-1][0].shape[1]
    Mb = min(M, 256)
    operands = [G, o]
    in_specs = [
        pl.BlockSpec((1, Mb, K, C1), lambda b, mb: (b, mb, 0, 0)),
        pl.BlockSpec((1, Mb, C1), lambda b, mb: (b, mb, 0)),
    ]
    for W, b in rest:
        operands.extend([W, b.reshape(1, -1)])
        in_specs.append(pl.BlockSpec(W.shape, lambda b_, mb: (0, 0)))
        in_specs.append(pl.BlockSpec((1, b.shape[0]), lambda b_, mb: (0, 0)))
    return pl.pallas_call(
        functools.partial(_sa_tail_kernel, K, len(rest)),
        grid=(B, M // Mb),
        in_specs=in_specs,
        out_specs=pl.BlockSpec((1, Mb, Cout), lambda b, mb: (b, mb, 0)),
        out_shape=jax.ShapeDtypeStruct((B, M, Cout), jnp.float32),
        interpret=interpret,
    )(*operands)


def _set_abstraction(xyz, feat, params, m, radius, K, interpret=False):
    cent_idx = _fps(xyz, m, interpret=interpret)
    new_xyz = _gather(xyz, cent_idx)
    nbr_idx = _ball_query(xyz, new_xyz, radius, K, interpret=interpret)
    W1, b1 = params[0]
    W1a = W1[:3]
    h = xyz @ W1a + b1
    if feat is not None:
        h = h + feat @ W1[3:]
    o = new_xyz @ W1a
    G = _gather(h, nbr_idx)
    new_feat = _sa_tail(G, o, params[1:], interpret=interpret)
    return new_xyz, new_feat


def _fp_kernel(Ns, k, n_layers, has_dense, *refs):
    dxyz = refs[0][0]
    sxyz_t = refs[1][0]
    sfeat = refs[2][0]
    pos = 3
    dfeat = None
    if has_dense:
        dfeat = refs[pos][0]
        pos += 1
    layers = []
    for _ in range(n_layers):
        layers.append((refs[pos], refs[pos + 1]))
        pos += 2
    o_ref = refs[pos]

    bb = jnp.sum(sxyz_t * sxyz_t, axis=0, keepdims=True)
    aa = jnp.sum(dxyz * dxyz, axis=1, keepdims=True)
    ab = jnp.dot(dxyz, sxyz_t, preferred_element_type=jnp.float32)
    d2 = jnp.maximum(aa + bb - 2.0 * ab, 0.0)
    iota_n = jax.lax.broadcasted_iota(jnp.int32, d2.shape, 1)
    d = d2
    dists, idxs = [], []
    for _ in range(k):
        mn = jnp.min(d, axis=1, keepdims=True)
        ik = jnp.min(jnp.where(d == mn, iota_n, Ns), axis=1, keepdims=True)
        dists.append(mn)
        idxs.append(ik)
        d = jnp.where(iota_n == ik, 1e30, d)
    ws = [1.0 / (mn + 1e-8) for mn in dists]
    denom = ws[0]
    for wk in ws[1:]:
        denom = denom + wk
    wmat = jnp.zeros(d2.shape, jnp.float32)
    for wk, ik in zip(ws, idxs):
        wmat = wmat + jnp.where(iota_n == ik, wk / denom, 0.0)
    x = jnp.dot(wmat, sfeat, preferred_element_type=jnp.float32)
    if dfeat is not None:
        x = jnp.concatenate([x, dfeat], axis=1)
    for W_ref, b_ref in layers:
        x = jnp.maximum(jnp.dot(x, W_ref[...], preferred_element_type=jnp.float32)
                        + b_ref[...], 0.0)
    o_ref[0] = x


def _feature_propagation(dense_xyz, sparse_xyz, dense_feat, sparse_feat, params, k,
                         interpret=False):
    dense_xyz = jax.lax.stop_gradient(dense_xyz)
    sparse_xyz = jax.lax.stop_gradient(sparse_xyz)
    B, Nd, _ = dense_xyz.shape
    Ns = sparse_xyz.shape[1]
    C = sparse_feat.shape[-1]
    Cout = params[-1][0].shape[1]
    Mb = min(Nd, 256)
    sxyz_t = jnp.transpose(sparse_xyz, (0, 2, 1))
    has_dense = dense_feat is not None
    operands = [dense_xyz, sxyz_t, sparse_feat]
    in_specs = [
        pl.BlockSpec((1, Mb, 3), lambda b, mb: (b, mb, 0)),
        pl.BlockSpec((1, 3, Ns), lambda b, mb: (b, 0, 0)),
        pl.BlockSpec((1, Ns, C), lambda b, mb: (b, 0, 0)),
    ]
    if has_dense:
        Cd = dense_feat.shape[-1]
        operands.append(dense_feat)
        in_specs.append(pl.BlockSpec((1, Mb, Cd), lambda b, mb: (b, mb, 0)))
    for W, b in params:
        operands.extend([W, b.reshape(1, -1)])
        in_specs.append(pl.BlockSpec(W.shape, lambda b_, mb: (0, 0)))
        in_specs.append(pl.BlockSpec((1, b.shape[0]), lambda b_, mb: (0, 0)))
    return pl.pallas_call(
        functools.partial(_fp_kernel, Ns, k, len(params), has_dense),
        grid=(B, Nd // Mb),
        in_specs=in_specs,
        out_specs=pl.BlockSpec((1, Mb, Cout), lambda b, mb: (b, mb, 0)),
        out_shape=jax.ShapeDtypeStruct((B, Nd, Cout), jnp.float32),
        interpret=interpret,
    )(*operands)


def _identity_pallas(x):
    def k(x_ref, o_ref):
        o_ref[...] = x_ref[...]
    return pl.pallas_call(
        k, out_shape=jax.ShapeDtypeStruct(x.shape, x.dtype))(x)


def kernel(points, sa_params, fp_params):
    xyz = jnp.transpose(points, (0, 2, 1))
    feat = None
    xyz_list = [xyz]
    feat_list = [None]
    for i in range(len(_SA_CHANNELS)):
        xyz, feat = _set_abstraction(xyz, feat, sa_params[i], _NUM_CENTROIDS[i],
                                     _RADIUS[i], _MAX_NEIGHBORS[i])
        xyz_list.append(xyz)
        feat_list.append(feat)
    fp_feat = feat_list[-1]
    for i in range(len(_FP_CHANNELS)):
        fp_feat = _feature_propagation(xyz_list[-2 - i], xyz_list[-1 - i],
                                       feat_list[-2 - i], fp_feat, fp_params[i],
                                       _FP_NEIGHBORS[i])
    fp_feat = _identity_pallas(fp_feat)
    return jnp.transpose(fp_feat, (0, 2, 1))
